# baseline (speedup 1.0000x reference)
import jax
import jax.numpy as jnp
from jax import lax
from jax.experimental import pallas as pl
from jax.experimental.pallas import tpu as pltpu
from jax.experimental.pallas import tpu_sc as plsc

VOCAB = 1000000
EMBED = 64
BATCH = 16384
K = 21
NC = 2
NS = 16
NW = NC * NS
EPW = BATCH // NW
CHUNK = 64
NCHUNK = EPW // CHUNK
NCT = NW * NCHUNK
KC = K * CHUNK
L = 16


def _sc_body(center_hbm, oidx_hbm, in_embed_hbm, out_embed_hbm, scores_hbm,
             cidx_v, oidx_v, crows_v, orows_v, scores_v, sem_i, sem_c, sem_o):
    cid = lax.axis_index("c")
    sid = lax.axis_index("s")
    wid = sid * NC + cid

    lane = lax.iota(jnp.int32, L)

    def chunk_body(i, carry):
        chunk = wid * NCHUNK + i
        base = wid * EPW + i * CHUNK
        i1 = pltpu.async_copy(center_hbm.at[pl.ds(base, CHUNK)], cidx_v, sem_i)
        i2 = pltpu.async_copy(oidx_hbm.at[chunk], oidx_v, sem_i)
        i1.wait()
        i2.wait()
        g1 = pltpu.async_copy(in_embed_hbm.at[cidx_v], crows_v, sem_c)
        g2 = pltpu.async_copy(out_embed_hbm.at[oidx_v], orows_v, sem_o)
        g1.wait()
        g2.wait()

        for g in range(CHUNK // L):
            e_idx = lane + (g * L)
            rows = [lane + (k * CHUNK + g * L) for k in range(K)]

            def d_body(d, accs):
                dv = jnp.zeros((L,), jnp.int32) + d
                cv = plsc.load_gather(crows_v, [e_idx, dv])
                return tuple(
                    accs[k] + plsc.load_gather(orows_v, [rows[k], dv]) * cv
                    for k in range(K)
                )

            accs = plsc.parallel_loop(
                0, EMBED, 1, unroll=4,
                carry=tuple(jnp.zeros((L,), jnp.float32) for _ in range(K)),
            )(d_body)
            for k in range(K):
                scores_v[pl.ds(k * CHUNK + g * L, L)] = accs[k]

        pltpu.sync_copy(scores_v, scores_hbm.at[chunk])
        return carry

    lax.fori_loop(0, NCHUNK, chunk_body, 0)


def _sc_scores(center, oidx, in_embed, out_embed):
    mesh = plsc.VectorSubcoreMesh(core_axis_name="c", subcore_axis_name="s")
    return pl.kernel(
        _sc_body,
        out_type=jax.ShapeDtypeStruct((NCT, KC), jnp.float32),
        mesh=mesh,
        compiler_params=pltpu.CompilerParams(
            use_tc_tiling_on_sc=False,
            needs_layout_passes=False,
        ),
        scratch_types=[
            pltpu.VMEM((CHUNK,), jnp.int32),
            pltpu.VMEM((KC,), jnp.int32),
            pltpu.VMEM((CHUNK, EMBED), jnp.float32),
            pltpu.VMEM((KC, EMBED), jnp.float32),
            pltpu.VMEM((KC,), jnp.float32),
            pltpu.SemaphoreType.DMA,
            pltpu.SemaphoreType.DMA,
            pltpu.SemaphoreType.DMA,
        ],
    )(center, oidx, in_embed, out_embed)


def _loss_body(s_ref, o_ref):
    s = s_ref[...]
    pos = s[:, :CHUNK]
    neg = s[:, CHUNK:]
    total = (jnp.sum(jnp.log(jax.nn.sigmoid(pos))) +
             jnp.sum(jnp.log(jax.nn.sigmoid(-neg))))
    o_ref[...] = jnp.reshape(-total / BATCH, (1, 1))


def _loss(scores):
    out = pl.pallas_call(
        _loss_body,
        out_shape=jax.ShapeDtypeStruct((1, 1), jnp.float32),
    )(scores)
    return out[0, 0]


@jax.jit
def kernel(center, context, negative, in_embed, out_embed):
    comb = jnp.concatenate([context[None, :], negative.T], axis=0)
    oidx = (comb.reshape(K, NCT, CHUNK).transpose(1, 0, 2).reshape(NCT, KC))
    scores = _sc_scores(center, oidx, in_embed, out_embed)
    return _loss(scores)


# speedup vs baseline: 1.3157x; 1.3157x over previous
import jax
import jax.numpy as jnp
from jax import lax
from jax.experimental import pallas as pl
from jax.experimental.pallas import tpu as pltpu
from jax.experimental.pallas import tpu_sc as plsc

VOCAB = 1000000
EMBED = 64
BATCH = 16384
NEG = 20
K = NEG + 1
NC = 2
NS = 16
NW = NC * NS
EPW = BATCH // NW
CHUNK = 64
NCHUNK = EPW // CHUNK
NCT = NW * NCHUNK
KC = K * CHUNK
L = 16
DJ = EMBED // L


def _sc_body(center_hbm, context_hbm, negt_hbm, in_embed_hbm, out_embed_hbm,
             scores_hbm, cidx_v, ctx_v, nidx_v, crows_v, orows_v, scores_v,
             sem_i, sem_c, sem_o):
    cid = lax.axis_index("c")
    sid = lax.axis_index("s")
    wid = sid * NC + cid

    lane = lax.iota(jnp.int32, L)

    def chunk_body(i, carry):
        chunk = wid * NCHUNK + i
        base = wid * EPW + i * CHUNK
        i1 = pltpu.async_copy(center_hbm.at[pl.ds(base, CHUNK)], cidx_v, sem_i)
        i2 = pltpu.async_copy(context_hbm.at[pl.ds(base, CHUNK)], ctx_v, sem_i)
        i3 = pltpu.async_copy(negt_hbm.at[:, pl.ds(base, CHUNK)], nidx_v,
                              sem_i)
        i1.wait()
        i2.wait()
        i3.wait()
        gdmas = [
            pltpu.async_copy(in_embed_hbm.at[cidx_v], crows_v, sem_c),
            pltpu.async_copy(out_embed_hbm.at[ctx_v], orows_v.at[0], sem_o),
        ]
        for kk in range(NEG):
            gdmas.append(
                pltpu.async_copy(out_embed_hbm.at[nidx_v.at[kk]],
                                 orows_v.at[kk + 1], sem_o))
        for g in gdmas:
            g.wait()

        for g in range(CHUNK // L):
            def e_body(el, accs):
                e = g * L + el
                msk = lane == el
                cs = [crows_v[e, pl.ds(j * L, L)] for j in range(DJ)]
                out = []
                for k in range(K):
                    p = cs[0] * orows_v[k, e, pl.ds(0, L)]
                    for j in range(1, DJ):
                        p = p + cs[j] * orows_v[k, e, pl.ds(j * L, L)]
                    out.append(accs[k] + jnp.where(msk, jnp.sum(p), 0.0))
                return tuple(out)

            accs = plsc.parallel_loop(
                0, L, 1, unroll=2,
                carry=tuple(jnp.zeros((L,), jnp.float32) for _ in range(K)),
            )(e_body)
            for k in range(K):
                scores_v[pl.ds(k * CHUNK + g * L, L)] = accs[k]

        pltpu.sync_copy(scores_v, scores_hbm.at[chunk])
        return carry

    lax.fori_loop(0, NCHUNK, chunk_body, 0)


def _sc_scores(center, context, negt, in_embed, out_embed):
    mesh = plsc.VectorSubcoreMesh(core_axis_name="c", subcore_axis_name="s")
    return pl.kernel(
        _sc_body,
        out_type=jax.ShapeDtypeStruct((NCT, KC), jnp.float32),
        mesh=mesh,
        compiler_params=pltpu.CompilerParams(
            use_tc_tiling_on_sc=False,
            needs_layout_passes=False,
        ),
        scratch_types=[
            pltpu.VMEM((CHUNK,), jnp.int32),
            pltpu.VMEM((CHUNK,), jnp.int32),
            pltpu.VMEM((NEG, CHUNK), jnp.int32),
            pltpu.VMEM((CHUNK, EMBED), jnp.float32),
            pltpu.VMEM((K, CHUNK, EMBED), jnp.float32),
            pltpu.VMEM((KC,), jnp.float32),
            pltpu.SemaphoreType.DMA,
            pltpu.SemaphoreType.DMA,
            pltpu.SemaphoreType.DMA,
        ],
    )(center, context, negt, in_embed, out_embed)


def _loss_body(s_ref, o_ref):
    s = s_ref[...]
    pos = s[:, :CHUNK]
    neg = s[:, CHUNK:]
    total = (jnp.sum(jnp.log(jax.nn.sigmoid(pos))) +
             jnp.sum(jnp.log(jax.nn.sigmoid(-neg))))
    o_ref[...] = jnp.reshape(-total / BATCH, (1, 1))


def _loss(scores):
    out = pl.pallas_call(
        _loss_body,
        out_shape=jax.ShapeDtypeStruct((1, 1), jnp.float32),
    )(scores)
    return out[0, 0]


@jax.jit
def kernel(center, context, negative, in_embed, out_embed):
    scores = _sc_scores(center, context, negative.T, in_embed, out_embed)
    return _loss(scores)


# speedup vs baseline: 1.3561x; 1.0306x over previous
import jax
import jax.numpy as jnp
from jax import lax
from jax.experimental import pallas as pl
from jax.experimental.pallas import tpu as pltpu
from jax.experimental.pallas import tpu_sc as plsc

VOCAB = 1000000
EMBED = 64
ROW = 128
BATCH = 16384
NEG = 20
K = NEG + 1
NC = 2
NS = 16
NW = NC * NS
EPW = BATCH // NW
CHUNK = 32
NCHUNK = EPW // CHUNK
NCT = NW * NCHUNK
KC = K * CHUNK
L = 16
DJ = EMBED // L


def _sc_body(center_hbm, context_hbm, negt_hbm, in_embed_hbm, out_embed_hbm,
             scores_hbm, cidx_v, ctx_v, nidx_v, crows_v, orows_v, scores_v,
             sem_i, sem_c, sem_o):
    cid = lax.axis_index("c")
    sid = lax.axis_index("s")
    wid = sid * NC + cid

    lane = lax.iota(jnp.int32, L)

    def chunk_body(i, carry):
        chunk = wid * NCHUNK + i
        base = wid * EPW + i * CHUNK
        i1 = pltpu.async_copy(center_hbm.at[pl.ds(base, CHUNK)], cidx_v, sem_i)
        i2 = pltpu.async_copy(context_hbm.at[pl.ds(base, CHUNK)], ctx_v, sem_i)
        i3 = pltpu.async_copy(negt_hbm.at[:, pl.ds(base, CHUNK)], nidx_v,
                              sem_i)
        i1.wait()
        i2.wait()
        i3.wait()
        gdmas = [
            pltpu.async_copy(in_embed_hbm.at[cidx_v], crows_v, sem_c),
            pltpu.async_copy(out_embed_hbm.at[ctx_v], orows_v.at[0], sem_o),
        ]
        for kk in range(NEG):
            gdmas.append(
                pltpu.async_copy(out_embed_hbm.at[nidx_v.at[kk]],
                                 orows_v.at[kk + 1], sem_o))
        for g in gdmas:
            g.wait()

        for g in range(CHUNK // L):
            def e_body(el, accs):
                e = g * L + el
                msk = lane == el
                cs = [crows_v[e, pl.ds(j * L, L)] for j in range(DJ)]
                out = []
                for k in range(K):
                    p = cs[0] * orows_v[k, e, pl.ds(0, L)]
                    for j in range(1, DJ):
                        p = p + cs[j] * orows_v[k, e, pl.ds(j * L, L)]
                    out.append(accs[k] + jnp.where(msk, jnp.sum(p), 0.0))
                return tuple(out)

            accs = plsc.parallel_loop(
                0, L, 1, unroll=2,
                carry=tuple(jnp.zeros((L,), jnp.float32) for _ in range(K)),
            )(e_body)
            for k in range(K):
                scores_v[pl.ds(k * CHUNK + g * L, L)] = accs[k]

        pltpu.sync_copy(scores_v, scores_hbm.at[chunk])
        return carry

    lax.fori_loop(0, NCHUNK, chunk_body, 0)


def _sc_scores(center, context, negt, in_embed, out_embed):
    mesh = plsc.VectorSubcoreMesh(core_axis_name="c", subcore_axis_name="s")
    return pl.kernel(
        _sc_body,
        out_type=jax.ShapeDtypeStruct((NCT, KC), jnp.float32),
        mesh=mesh,
        compiler_params=pltpu.CompilerParams(
            use_tc_tiling_on_sc=False,
            needs_layout_passes=False,
        ),
        scratch_types=[
            pltpu.VMEM((CHUNK,), jnp.int32),
            pltpu.VMEM((CHUNK,), jnp.int32),
            pltpu.VMEM((NEG, CHUNK), jnp.int32),
            pltpu.VMEM((CHUNK, ROW), jnp.float32),
            pltpu.VMEM((K, CHUNK, ROW), jnp.float32),
            pltpu.VMEM((KC,), jnp.float32),
            pltpu.SemaphoreType.DMA,
            pltpu.SemaphoreType.DMA,
            pltpu.SemaphoreType.DMA,
        ],
    )(center, context, negt, in_embed, out_embed)


def _loss_body(s_ref, o_ref):
    s = s_ref[...]
    pos = s[:, :CHUNK]
    neg = s[:, CHUNK:]
    total = (jnp.sum(jnp.log(jax.nn.sigmoid(pos))) +
             jnp.sum(jnp.log(jax.nn.sigmoid(-neg))))
    o_ref[...] = jnp.reshape(-total / BATCH, (1, 1))


def _loss(scores):
    out = pl.pallas_call(
        _loss_body,
        out_shape=jax.ShapeDtypeStruct((1, 1), jnp.float32),
    )(scores)
    return out[0, 0]


@jax.jit
def kernel(center, context, negative, in_embed, out_embed):
    in_p = jnp.pad(in_embed, ((0, 0), (0, ROW - EMBED)))
    out_p = jnp.pad(out_embed, ((0, 0), (0, ROW - EMBED)))
    scores = _sc_scores(center, context, negative.T, in_p, out_p)
    return _loss(scores)


# speedup vs baseline: 1.3952x; 1.0288x over previous
import jax
import jax.numpy as jnp
from jax import lax
from jax.experimental import pallas as pl
from jax.experimental.pallas import tpu as pltpu
from jax.experimental.pallas import tpu_sc as plsc

VOCAB = 1000000
EMBED = 64
ROW = 64
BATCH = 16384
NEG = 20
K = NEG + 1
NC = 2
NS = 16
NW = NC * NS
EPW = BATCH // NW
CHUNK = 64
NCHUNK = EPW // CHUNK
NCT = NW * NCHUNK
KC = K * CHUNK
L = 16
DJ = EMBED // L


def _sc_body(center_hbm, context_hbm, negt_hbm, in_embed_hbm, out_embed_hbm,
             scores_hbm, cidx_v, ctx_v, nidx_v, crows_v, orows_v, scores_v,
             sem_i, sem_c, sem_o):
    cid = lax.axis_index("c")
    sid = lax.axis_index("s")
    wid = sid * NC + cid

    lane = lax.iota(jnp.int32, L)

    def chunk_body(i, carry):
        chunk = wid * NCHUNK + i
        base = wid * EPW + i * CHUNK
        i1 = pltpu.async_copy(center_hbm.at[pl.ds(base, CHUNK)], cidx_v, sem_i)
        i2 = pltpu.async_copy(context_hbm.at[pl.ds(base, CHUNK)], ctx_v, sem_i)
        i3 = pltpu.async_copy(negt_hbm.at[:, pl.ds(base, CHUNK)], nidx_v,
                              sem_i)
        i1.wait()
        i2.wait()
        i3.wait()
        for s in range(CHUNK // L):
            cidx_v[pl.ds(s * L, L)] = cidx_v[pl.ds(s * L, L)] * 2
            ctx_v[pl.ds(s * L, L)] = ctx_v[pl.ds(s * L, L)] * 2
        for kk in range(NEG):
            for s in range(CHUNK // L):
                nidx_v[kk, pl.ds(s * L, L)] = nidx_v[kk, pl.ds(s * L, L)] * 2
        gdmas = [
            pltpu.async_copy(in_embed_hbm.at[cidx_v], crows_v, sem_c),
            pltpu.async_copy(out_embed_hbm.at[ctx_v], orows_v.at[0], sem_o),
        ]
        for kk in range(NEG):
            gdmas.append(
                pltpu.async_copy(out_embed_hbm.at[nidx_v.at[kk]],
                                 orows_v.at[kk + 1], sem_o))
        for g in gdmas:
            g.wait()

        for g in range(CHUNK // L):
            def e_body(el, accs):
                e = g * L + el
                msk = lane == el
                cs = [crows_v[e, pl.ds(j * L, L)] for j in range(DJ)]
                out = []
                for k in range(K):
                    p = cs[0] * orows_v[k, e, pl.ds(0, L)]
                    for j in range(1, DJ):
                        p = p + cs[j] * orows_v[k, e, pl.ds(j * L, L)]
                    out.append(accs[k] + jnp.where(msk, jnp.sum(p), 0.0))
                return tuple(out)

            accs = plsc.parallel_loop(
                0, L, 1, unroll=2,
                carry=tuple(jnp.zeros((L,), jnp.float32) for _ in range(K)),
            )(e_body)
            for k in range(K):
                scores_v[pl.ds(k * CHUNK + g * L, L)] = accs[k]

        pltpu.sync_copy(scores_v, scores_hbm.at[chunk])
        return carry

    lax.fori_loop(0, NCHUNK, chunk_body, 0)


def _sc_scores(center, context, negt, in_embed, out_embed):
    mesh = plsc.VectorSubcoreMesh(core_axis_name="c", subcore_axis_name="s")
    return pl.kernel(
        _sc_body,
        out_type=jax.ShapeDtypeStruct((NCT, KC), jnp.float32),
        mesh=mesh,
        compiler_params=pltpu.CompilerParams(
            use_tc_tiling_on_sc=False,
            needs_layout_passes=False,
        ),
        scratch_types=[
            pltpu.VMEM((CHUNK,), jnp.int32),
            pltpu.VMEM((CHUNK,), jnp.int32),
            pltpu.VMEM((NEG, CHUNK), jnp.int32),
            pltpu.VMEM((CHUNK, ROW), jnp.float32),
            pltpu.VMEM((K, CHUNK, ROW), jnp.float32),
            pltpu.VMEM((KC,), jnp.float32),
            pltpu.SemaphoreType.DMA,
            pltpu.SemaphoreType.DMA,
            pltpu.SemaphoreType.DMA,
        ],
    )(center, context, negt, in_embed, out_embed)


def _loss_body(s_ref, o_ref):
    s = s_ref[...]
    pos = s[:, :CHUNK]
    neg = s[:, CHUNK:]
    total = (jnp.sum(jnp.log(jax.nn.sigmoid(pos))) +
             jnp.sum(jnp.log(jax.nn.sigmoid(-neg))))
    o_ref[...] = jnp.reshape(-total / BATCH, (1, 1))


def _loss(scores):
    out = pl.pallas_call(
        _loss_body,
        out_shape=jax.ShapeDtypeStruct((1, 1), jnp.float32),
    )(scores)
    return out[0, 0]


@jax.jit
def kernel(center, context, negative, in_embed, out_embed):
    in_p = jnp.pad(in_embed, ((0, 0), (0, EMBED))).reshape(2 * VOCAB, EMBED)
    out_p = jnp.pad(out_embed, ((0, 0), (0, EMBED))).reshape(2 * VOCAB, EMBED)
    scores = _sc_scores(center, context, negative.T, in_p, out_p)
    return _loss(scores)


# speedup vs baseline: 2.8413x; 2.0366x over previous
import jax
import jax.numpy as jnp
from jax import lax
from jax.experimental import pallas as pl
from jax.experimental.pallas import tpu as pltpu
from jax.experimental.pallas import tpu_sc as plsc

VOCAB = 1000000
EMBED = 64
ROW = 64
BATCH = 16384
NEG = 20
K = NEG + 1
NC = 2
NS = 16
NW = NC * NS
EPW = BATCH // NW
CHUNK = 64
NCHUNK = EPW // CHUNK
NCT = NW * NCHUNK
KC = K * CHUNK
L = 16
DJ = EMBED // L


def _sc_body(center_hbm, context_hbm, negt_hbm, emb_hbm,
             scores_hbm, cidx_v, ctx_v, nidx_v, crows_v, orows_v, scores_v,
             sem_i, sem_c, sem_o):
    cid = lax.axis_index("c")
    sid = lax.axis_index("s")
    wid = sid * NC + cid

    lane = lax.iota(jnp.int32, L)

    def chunk_body(i, carry):
        chunk = wid * NCHUNK + i
        base = wid * EPW + i * CHUNK
        i1 = pltpu.async_copy(center_hbm.at[pl.ds(base, CHUNK)], cidx_v, sem_i)
        i2 = pltpu.async_copy(context_hbm.at[pl.ds(base, CHUNK)], ctx_v, sem_i)
        i3 = pltpu.async_copy(negt_hbm.at[:, pl.ds(base, CHUNK)], nidx_v,
                              sem_i)
        i1.wait()
        i2.wait()
        i3.wait()
        for s in range(CHUNK // L):
            cidx_v[pl.ds(s * L, L)] = cidx_v[pl.ds(s * L, L)] * 2
            ctx_v[pl.ds(s * L, L)] = ctx_v[pl.ds(s * L, L)] * 2 + 1
        for kk in range(NEG):
            for s in range(CHUNK // L):
                nidx_v[kk, pl.ds(s * L, L)] = (
                    nidx_v[kk, pl.ds(s * L, L)] * 2 + 1)
        gdmas = [
            pltpu.async_copy(emb_hbm.at[cidx_v], crows_v, sem_c),
            pltpu.async_copy(emb_hbm.at[ctx_v], orows_v.at[0], sem_o),
        ]
        for kk in range(NEG):
            gdmas.append(
                pltpu.async_copy(emb_hbm.at[nidx_v.at[kk]],
                                 orows_v.at[kk + 1], sem_o))
        for g in gdmas:
            g.wait()

        for g in range(CHUNK // L):
            def e_body(el, accs):
                e = g * L + el
                msk = lane == el
                cs = [crows_v[e, pl.ds(j * L, L)] for j in range(DJ)]
                out = []
                for k in range(K):
                    p = cs[0] * orows_v[k, e, pl.ds(0, L)]
                    for j in range(1, DJ):
                        p = p + cs[j] * orows_v[k, e, pl.ds(j * L, L)]
                    out.append(accs[k] + jnp.where(msk, jnp.sum(p), 0.0))
                return tuple(out)

            accs = plsc.parallel_loop(
                0, L, 1, unroll=2,
                carry=tuple(jnp.zeros((L,), jnp.float32) for _ in range(K)),
            )(e_body)
            for k in range(K):
                scores_v[pl.ds(k * CHUNK + g * L, L)] = accs[k]

        pltpu.sync_copy(scores_v, scores_hbm.at[chunk])
        return carry

    lax.fori_loop(0, NCHUNK, chunk_body, 0)


def _sc_scores(center, context, negt, emb):
    mesh = plsc.VectorSubcoreMesh(core_axis_name="c", subcore_axis_name="s")
    return pl.kernel(
        _sc_body,
        out_type=jax.ShapeDtypeStruct((NCT, KC), jnp.float32),
        mesh=mesh,
        compiler_params=pltpu.CompilerParams(
            use_tc_tiling_on_sc=False,
            needs_layout_passes=False,
        ),
        scratch_types=[
            pltpu.VMEM((CHUNK,), jnp.int32),
            pltpu.VMEM((CHUNK,), jnp.int32),
            pltpu.VMEM((NEG, CHUNK), jnp.int32),
            pltpu.VMEM((CHUNK, ROW), jnp.float32),
            pltpu.VMEM((K, CHUNK, ROW), jnp.float32),
            pltpu.VMEM((KC,), jnp.float32),
            pltpu.SemaphoreType.DMA,
            pltpu.SemaphoreType.DMA,
            pltpu.SemaphoreType.DMA,
        ],
    )(center, context, negt, emb)


FW = 8192
FGRID = (VOCAB + FW - 1) // FW


def _fmt_body(xa_ref, xb_ref, y_ref):
    y_ref[...] = jnp.concatenate([xa_ref[...].T, xb_ref[...].T], axis=1)


def _fmt(in_t, out_t):
    y = pl.pallas_call(
        _fmt_body,
        grid=(FGRID,),
        in_specs=[pl.BlockSpec((EMBED, FW), lambda i: (0, i)),
                  pl.BlockSpec((EMBED, FW), lambda i: (0, i))],
        out_specs=pl.BlockSpec((FW, 2 * EMBED), lambda i: (i, 0)),
        out_shape=jax.ShapeDtypeStruct((VOCAB, 2 * EMBED), jnp.float32),
    )(in_t, out_t)
    return y.reshape(2 * VOCAB, EMBED)


def _loss_body(s_ref, o_ref):
    s = s_ref[...]
    pos = s[:, :CHUNK]
    neg = s[:, CHUNK:]
    total = (jnp.sum(jnp.log(jax.nn.sigmoid(pos))) +
             jnp.sum(jnp.log(jax.nn.sigmoid(-neg))))
    o_ref[...] = jnp.reshape(-total / BATCH, (1, 1))


def _loss(scores):
    out = pl.pallas_call(
        _loss_body,
        out_shape=jax.ShapeDtypeStruct((1, 1), jnp.float32),
    )(scores)
    return out[0, 0]


@jax.jit
def kernel(center, context, negative, in_embed, out_embed):
    emb = _fmt(in_embed.T, out_embed.T)
    scores = _sc_scores(center, context, negative.T, emb)
    return _loss(scores)


# speedup vs baseline: 3.0144x; 1.0609x over previous
import jax
import jax.numpy as jnp
from jax import lax
from jax.experimental import pallas as pl
from jax.experimental.pallas import tpu as pltpu
from jax.experimental.pallas import tpu_sc as plsc

VOCAB = 1000000
EMBED = 64
ROW = 64
BATCH = 16384
NEG = 20
K = NEG + 1
NC = 2
NS = 16
NW = NC * NS
EPW = BATCH // NW
CHUNK = 64
NCHUNK = EPW // CHUNK
NCT = NW * NCHUNK
KC = K * CHUNK
L = 16
DJ = EMBED // L


def _sc_body(center_hbm, context_hbm, negt_hbm, emb_hbm,
             scores_hbm, cidx_v, ctx_v, nidx_v, crows_v, orows_v, scores_v,
             sem_i, sem_c, sem_o):
    cid = lax.axis_index("c")
    sid = lax.axis_index("s")
    wid = sid * NC + cid

    lane = lax.iota(jnp.int32, L)

    def chunk_body(i, carry):
        chunk = wid * NCHUNK + i
        base = wid * EPW + i * CHUNK
        i1 = pltpu.async_copy(center_hbm.at[pl.ds(base, CHUNK)], cidx_v, sem_i)
        i2 = pltpu.async_copy(context_hbm.at[pl.ds(base, CHUNK)], ctx_v, sem_i)
        i3 = pltpu.async_copy(negt_hbm.at[:, pl.ds(base, CHUNK)], nidx_v,
                              sem_i)
        i1.wait()
        i2.wait()
        i3.wait()
        for s in range(CHUNK // L):
            cidx_v[pl.ds(s * L, L)] = cidx_v[pl.ds(s * L, L)] * 2
            ctx_v[pl.ds(s * L, L)] = ctx_v[pl.ds(s * L, L)] * 2 + 1
        for kk in range(NEG):
            for s in range(CHUNK // L):
                nidx_v[kk, pl.ds(s * L, L)] = (
                    nidx_v[kk, pl.ds(s * L, L)] * 2 + 1)
        gdmas = [
            pltpu.async_copy(emb_hbm.at[cidx_v], crows_v, sem_c),
            pltpu.async_copy(emb_hbm.at[ctx_v], orows_v.at[0], sem_o),
        ]
        for kk in range(NEG):
            gdmas.append(
                pltpu.async_copy(emb_hbm.at[nidx_v.at[kk]],
                                 orows_v.at[kk + 1], sem_o))
        for g in gdmas:
            g.wait()

        for g in range(CHUNK // L):
            def e_body(el, accs):
                e = g * L + el
                msk = lane == el
                cs = [crows_v[e, pl.ds(j * L, L)] for j in range(DJ)]
                out = []
                for k in range(K):
                    p = cs[0] * orows_v[k, e, pl.ds(0, L)]
                    for j in range(1, DJ):
                        p = p + cs[j] * orows_v[k, e, pl.ds(j * L, L)]
                    out.append(accs[k] + jnp.where(msk, jnp.sum(p), 0.0))
                return tuple(out)

            accs = plsc.parallel_loop(
                0, L, 1, unroll=2,
                carry=tuple(jnp.zeros((L,), jnp.float32) for _ in range(K)),
            )(e_body)
            for k in range(K):
                scores_v[pl.ds(k * CHUNK + g * L, L)] = accs[k]

        pltpu.sync_copy(scores_v, scores_hbm.at[chunk])
        return carry

    lax.fori_loop(0, NCHUNK, chunk_body, 0)


def _sc_scores(center, context, negt, emb):
    mesh = plsc.VectorSubcoreMesh(core_axis_name="c", subcore_axis_name="s")
    return pl.kernel(
        _sc_body,
        out_type=jax.ShapeDtypeStruct((NCT, KC), jnp.float32),
        mesh=mesh,
        compiler_params=pltpu.CompilerParams(
            use_tc_tiling_on_sc=False,
            needs_layout_passes=False,
        ),
        scratch_types=[
            pltpu.VMEM((CHUNK,), jnp.int32),
            pltpu.VMEM((CHUNK,), jnp.int32),
            pltpu.VMEM((NEG, CHUNK), jnp.int32),
            pltpu.VMEM((CHUNK, ROW), jnp.float32),
            pltpu.VMEM((K, CHUNK, ROW), jnp.float32),
            pltpu.VMEM((KC,), jnp.float32),
            pltpu.SemaphoreType.DMA,
            pltpu.SemaphoreType.DMA,
            pltpu.SemaphoreType.DMA,
        ],
    )(center, context, negt, emb)


FW = 16384
FGRID = (VOCAB + FW - 1) // FW


def _fmt_body(xa_ref, xb_ref, y_ref):
    y_ref[...] = jnp.concatenate([xa_ref[...].T, xb_ref[...].T], axis=1)


def _fmt(in_t, out_t):
    y = pl.pallas_call(
        _fmt_body,
        grid=(FGRID,),
        in_specs=[pl.BlockSpec((EMBED, FW), lambda i: (0, i)),
                  pl.BlockSpec((EMBED, FW), lambda i: (0, i))],
        out_specs=pl.BlockSpec((FW, 2 * EMBED), lambda i: (i, 0)),
        out_shape=jax.ShapeDtypeStruct((VOCAB, 2 * EMBED), jnp.float32),
    )(in_t, out_t)
    return y.reshape(2 * VOCAB, EMBED)


def _loss_body(s_ref, o_ref):
    s = s_ref[...]
    pos = s[:, :CHUNK]
    neg = s[:, CHUNK:]
    total = (jnp.sum(jnp.log(jax.nn.sigmoid(pos))) +
             jnp.sum(jnp.log(jax.nn.sigmoid(-neg))))
    o_ref[...] = jnp.reshape(-total / BATCH, (1, 1))


def _loss(scores):
    out = pl.pallas_call(
        _loss_body,
        out_shape=jax.ShapeDtypeStruct((1, 1), jnp.float32),
    )(scores)
    return out[0, 0]


@jax.jit
def kernel(center, context, negative, in_embed, out_embed):
    emb = _fmt(in_embed.T, out_embed.T)
    scores = _sc_scores(center, context, negative.T, emb)
    return _loss(scores)


# speedup vs baseline: 3.0193x; 1.0016x over previous
import jax
import jax.numpy as jnp
from jax import lax
from jax.experimental import pallas as pl
from jax.experimental.pallas import tpu as pltpu
from jax.experimental.pallas import tpu_sc as plsc

VOCAB = 1000000
EMBED = 64
ROW = 64
BATCH = 16384
NEG = 20
K = NEG + 1
NC = 2
NS = 16
NW = NC * NS
EPW = BATCH // NW
CHUNK = 64
NCHUNK = EPW // CHUNK
NCT = NW * NCHUNK
KC = K * CHUNK
L = 16
DJ = EMBED // L


def _sc_body(center_hbm, context_hbm, negt_hbm, emb_hbm,
             scores_hbm, cidx_v, ctx_v, nidx_v, crows_v, orows_v, scores_v,
             sem_i, sem_c, sem_o):
    cid = lax.axis_index("c")
    sid = lax.axis_index("s")
    wid = sid * NC + cid

    lane = lax.iota(jnp.int32, L)

    def chunk_body(i, carry):
        chunk = wid * NCHUNK + i
        base = wid * EPW + i * CHUNK
        i1 = pltpu.async_copy(center_hbm.at[pl.ds(base, CHUNK)], cidx_v, sem_i)
        i2 = pltpu.async_copy(context_hbm.at[pl.ds(base, CHUNK)], ctx_v, sem_i)
        i3 = pltpu.async_copy(negt_hbm.at[:, pl.ds(base, CHUNK)], nidx_v,
                              sem_i)
        i1.wait()
        i2.wait()
        i3.wait()
        for s in range(CHUNK // L):
            cidx_v[pl.ds(s * L, L)] = cidx_v[pl.ds(s * L, L)] * 2
            ctx_v[pl.ds(s * L, L)] = ctx_v[pl.ds(s * L, L)] * 2 + 1
        for kk in range(NEG):
            for s in range(CHUNK // L):
                nidx_v[kk, pl.ds(s * L, L)] = (
                    nidx_v[kk, pl.ds(s * L, L)] * 2 + 1)
        gdmas = [
            pltpu.async_copy(emb_hbm.at[cidx_v], crows_v, sem_c),
            pltpu.async_copy(emb_hbm.at[ctx_v], orows_v.at[0], sem_o),
        ]
        for kk in range(NEG):
            gdmas.append(
                pltpu.async_copy(emb_hbm.at[nidx_v.at[kk]],
                                 orows_v.at[kk + 1], sem_o))
        for g in gdmas:
            g.wait()

        for g in range(CHUNK // L):
            def e_body(el, accs):
                e = g * L + el
                msk = lane == el
                cs = [crows_v[e, pl.ds(j * L, L)] for j in range(DJ)]
                out = []
                for k in range(K):
                    p = cs[0] * orows_v[k, e, pl.ds(0, L)]
                    for j in range(1, DJ):
                        p = p + cs[j] * orows_v[k, e, pl.ds(j * L, L)]
                    out.append(accs[k] + jnp.where(msk, jnp.sum(p), 0.0))
                return tuple(out)

            accs = plsc.parallel_loop(
                0, L, 1, unroll=2,
                carry=tuple(jnp.zeros((L,), jnp.float32) for _ in range(K)),
            )(e_body)
            for k in range(K):
                scores_v[pl.ds(k * CHUNK + g * L, L)] = accs[k]

        pltpu.sync_copy(scores_v, scores_hbm.at[chunk])
        return carry

    lax.fori_loop(0, NCHUNK, chunk_body, 0)


def _sc_scores(center, context, negt, emb):
    mesh = plsc.VectorSubcoreMesh(core_axis_name="c", subcore_axis_name="s")
    return pl.kernel(
        _sc_body,
        out_type=jax.ShapeDtypeStruct((NCT, KC), jnp.float32),
        mesh=mesh,
        compiler_params=pltpu.CompilerParams(
            use_tc_tiling_on_sc=False,
            needs_layout_passes=False,
        ),
        scratch_types=[
            pltpu.VMEM((CHUNK,), jnp.int32),
            pltpu.VMEM((CHUNK,), jnp.int32),
            pltpu.VMEM((NEG, CHUNK), jnp.int32),
            pltpu.VMEM((CHUNK, ROW), jnp.float32),
            pltpu.VMEM((K, CHUNK, ROW), jnp.float32),
            pltpu.VMEM((KC,), jnp.float32),
            pltpu.SemaphoreType.DMA,
            pltpu.SemaphoreType.DMA,
            pltpu.SemaphoreType.DMA,
        ],
    )(center, context, negt, emb)


FW = 16384
FGRID = (VOCAB + FW - 1) // FW


def _fmt_body(xa_ref, xb_ref, y_ref):
    y_ref[:, 0:EMBED] = xa_ref[...].T
    y_ref[:, EMBED:2 * EMBED] = xb_ref[...].T


def _fmt(in_t, out_t):
    y = pl.pallas_call(
        _fmt_body,
        grid=(FGRID,),
        in_specs=[pl.BlockSpec((EMBED, FW), lambda i: (0, i)),
                  pl.BlockSpec((EMBED, FW), lambda i: (0, i))],
        out_specs=pl.BlockSpec((FW, 2 * EMBED), lambda i: (i, 0)),
        out_shape=jax.ShapeDtypeStruct((VOCAB, 2 * EMBED), jnp.float32),
    )(in_t, out_t)
    return y.reshape(2 * VOCAB, EMBED)


def _loss_body(s_ref, o_ref):
    s = s_ref[...]
    pos = s[:, :CHUNK]
    neg = s[:, CHUNK:]
    total = (jnp.sum(jnp.log(jax.nn.sigmoid(pos))) +
             jnp.sum(jnp.log(jax.nn.sigmoid(-neg))))
    o_ref[...] = jnp.reshape(-total / BATCH, (1, 1))


def _loss(scores):
    out = pl.pallas_call(
        _loss_body,
        out_shape=jax.ShapeDtypeStruct((1, 1), jnp.float32),
    )(scores)
    return out[0, 0]


@jax.jit
def kernel(center, context, negative, in_embed, out_embed):
    emb = _fmt(in_embed.T, out_embed.T)
    scores = _sc_scores(center, context, negative.T, emb)
    return _loss(scores)
